# baseline (speedup 1.0000x reference)
import functools

import jax
import jax.numpy as jnp
from jax import lax
from jax.experimental import pallas as pl
from jax.experimental.pallas import tpu as pltpu

NS = 4
T = 8


def _body(x_ref, w1_ref, wt_ref, b1_ref, w2_ref, b2_ref, m6_ref,
          act_out, pos_out, rot_out, mask_out,
          act_b, pos_b, rot_b,
          carry_act, carry_pos, carry_rot,
          row0_act, row0_pos, row0_rot,
          cnt_ref, offs_ref, pend_ref,
          sem_act, sem_pos, sem_rot, sem_aux,
          *, R, G, M, F127):
    RB = R + T
    i = pl.program_id(0)
    slot = lax.rem(i, NS)

    @pl.when(i == 0)
    def _init():
        cnt_ref[0] = 0
        for s in range(NS):
            pend_ref[s] = 0

    def drain(s, off):
        off = pl.multiple_of(off, T)
        pltpu.make_async_copy(act_b.at[s, pl.ds(0, R)],
                              act_out.at[pl.ds(off, R)], sem_act.at[s]).wait()
        pltpu.make_async_copy(pos_b.at[s, pl.ds(0, R)],
                              pos_out.at[pl.ds(off, R)], sem_pos.at[s]).wait()
        pltpu.make_async_copy(rot_b.at[s, pl.ds(0, R)],
                              rot_out.at[pl.ds(off, R)], sem_rot.at[s]).wait()

    @pl.when(pend_ref[slot] == 1)
    def _reclaim():
        drain(slot, offs_ref[slot])
        pend_ref[slot] = 0

    x = x_ref[...]
    maskf = (x[:, F127:F127 + 1] != 0.0).astype(jnp.float32)
    mask_out[...] = maskf

    goal = x[:, F127 - 2:F127]
    posv = x[:, F127 + 1:F127 + 3]
    diff = goal - posv
    d2 = diff * diff
    ones22 = jnp.full((2, 2), 1.0, dtype=jnp.float32)
    r2p = jnp.dot(d2, ones22, preferred_element_type=jnp.float32)
    inv = lax.rsqrt(r2p)
    zero = r2p == 0.0
    lane0 = (lax.broadcasted_iota(jnp.int32, diff.shape, 1) == 0)
    cs = jnp.where(zero, jnp.where(lane0, 1.0, 0.0),
                   diff * jnp.where(zero, 0.0, inv))
    i22 = lax.broadcasted_iota(jnp.int32, (2, 2), 0)
    j22 = lax.broadcasted_iota(jnp.int32, (2, 2), 1)
    swapm = (i22 != j22).astype(jnp.float32)
    sw = jnp.dot(cs, swapm, preferred_element_type=jnp.float32)
    u = cs * diff
    w = sw * diff
    uwcs = jnp.concatenate([u, w, cs], axis=1)
    out6 = jnp.dot(uwcs, m6_ref[...],
                   preferred_element_type=jnp.float32)
    lgpair = out6[:, 0:2]
    rotv = out6[:, 2:6]

    h = jnp.maximum(
        jnp.dot(x[:, :F127 + 1], w1_ref[...],
                preferred_element_type=jnp.float32)
        + jnp.dot(lgpair, wt_ref[...], preferred_element_type=jnp.float32)
        + b1_ref[...], 0.0)
    act = (jnp.dot(h, w2_ref[...], preferred_element_type=jnp.float32)
           + b2_ref[...])

    nblk = jnp.sum(maskf).astype(jnp.int32)
    cnt = cnt_ref[0]
    base = pl.multiple_of((cnt // T) * T, T)
    shift = cnt - base
    cnt2 = cnt + nblk
    base2 = pl.multiple_of((cnt2 // T) * T, T)

    fast = jnp.logical_and(nblk == R, shift == 0)

    @pl.when(fast)
    def _fast():
        act_b[slot, pl.ds(0, R)] = act
        pos_b[slot, pl.ds(0, R)] = posv
        rot_b[slot, pl.ds(0, R)] = rotv

    @pl.when(jnp.logical_not(fast))
    def _pack():
        ii = lax.broadcasted_iota(jnp.int32, (R, R), 0)
        jj = lax.broadcasted_iota(jnp.int32, (R, R), 1)
        eye = (ii == jj).astype(jnp.float32)
        su = (ii < jj).astype(jnp.float32)
        diag_m = eye * jnp.broadcast_to(maskf, (R, R))
        onesb = jnp.full((RB, R), 1.0, dtype=jnp.float32)
        mask_lane = jnp.dot(onesb, diag_m,
                            preferred_element_type=jnp.float32)
        cum_excl = jnp.dot(mask_lane, su,
                           preferred_element_type=jnp.float32)
        pio = lax.broadcasted_iota(jnp.int32, (RB, R), 0).astype(jnp.float32)
        shf = shift.astype(jnp.float32)
        P = jnp.where(cum_excl + shf == pio, mask_lane, 0.0)

        cmask = lax.broadcasted_iota(jnp.int32, (T, 1), 0) < shift
        zpad_a = jnp.zeros((R, act.shape[1]), jnp.float32)
        zpad_p = jnp.zeros((R, 2), jnp.float32)
        zpad_r = jnp.zeros((R, 4), jnp.float32)
        ca = jnp.concatenate(
            [jnp.where(cmask, carry_act[...], 0.0), zpad_a], axis=0)
        cp = jnp.concatenate(
            [jnp.where(cmask, carry_pos[...], 0.0), zpad_p], axis=0)
        cr = jnp.concatenate(
            [jnp.where(cmask, carry_rot[...], 0.0), zpad_r], axis=0)
        act_b[slot] = jnp.dot(P, act,
                              preferred_element_type=jnp.float32) + ca
        pos_b[slot] = jnp.dot(P, posv,
                              preferred_element_type=jnp.float32) + cp
        rot_b[slot] = jnp.dot(P, rotv,
                              preferred_element_type=jnp.float32) + cr

    pltpu.make_async_copy(act_b.at[slot, pl.ds(0, R)],
                          act_out.at[pl.ds(base, R)], sem_act.at[slot]).start()
    pltpu.make_async_copy(pos_b.at[slot, pl.ds(0, R)],
                          pos_out.at[pl.ds(base, R)], sem_pos.at[slot]).start()
    pltpu.make_async_copy(rot_b.at[slot, pl.ds(0, R)],
                          rot_out.at[pl.ds(base, R)], sem_rot.at[slot]).start()
    offs_ref[slot] = base
    pend_ref[slot] = 1
    cnt_ref[0] = cnt2

    @pl.when(jnp.logical_not(fast))
    def _after_pack():
        d = pl.multiple_of(base2 - base, T)

        @pl.when(cnt2 - base2 > 0)
        def _extract():
            ea = pltpu.make_async_copy(act_b.at[slot, pl.ds(d, T)], carry_act,
                                       sem_aux)
            ep = pltpu.make_async_copy(pos_b.at[slot, pl.ds(d, T)], carry_pos,
                                       sem_aux)
            er = pltpu.make_async_copy(rot_b.at[slot, pl.ds(d, T)], carry_rot,
                                       sem_aux)
            ea.start()
            ea.wait()
            ep.start()
            ep.wait()
            er.start()
            er.wait()

        drain(slot, base)
        pend_ref[slot] = 0

    @pl.when(i == 0)
    def _save_row0():
        row0_act[...] = jnp.broadcast_to(act[0:1, :], (T, act.shape[1]))
        row0_pos[...] = jnp.broadcast_to(posv[0:1, :], (T, 2))
        row0_rot[...] = jnp.broadcast_to(rotv[0:1, :], (T, 4))

    @pl.when(i == G - 1)
    def _tail():
        for sconst in range(NS):
            @pl.when(pend_ref[sconst] == 1)
            def _d(sconst=sconst):
                drain(sconst, offs_ref[sconst])
                pend_ref[sconst] = 0

        cfinal = cnt_ref[0]
        basef = pl.multiple_of((cfinal // T) * T, T)
        shiftf = cfinal - basef

        @pl.when(shiftf > 0)
        def _flush():
            fmask = lax.broadcasted_iota(jnp.int32, (T, 1), 0) < shiftf
            carry_act[...] = jnp.where(fmask, carry_act[...], row0_act[...])
            carry_pos[...] = jnp.where(fmask, carry_pos[...], row0_pos[...])
            carry_rot[...] = jnp.where(fmask, carry_rot[...], row0_rot[...])
            pltpu.make_async_copy(carry_act, act_out.at[pl.ds(basef, T)],
                                  sem_act.at[0]).start()
            pltpu.make_async_copy(carry_pos, pos_out.at[pl.ds(basef, T)],
                                  sem_pos.at[0]).start()
            pltpu.make_async_copy(carry_rot, rot_out.at[pl.ds(basef, T)],
                                  sem_rot.at[0]).start()
            pltpu.make_async_copy(carry_act, act_out.at[pl.ds(basef, T)],
                                  sem_act.at[0]).wait()
            pltpu.make_async_copy(carry_pos, pos_out.at[pl.ds(basef, T)],
                                  sem_pos.at[0]).wait()
            pltpu.make_async_copy(carry_rot, rot_out.at[pl.ds(basef, T)],
                                  sem_rot.at[0]).wait()

        start = pl.multiple_of(basef + jnp.where(shiftf > 0, T, 0), T)

        def fill(k, carry):
            off = pl.multiple_of(start + k * T, T)
            fa = pltpu.make_async_copy(row0_act, act_out.at[pl.ds(off, T)],
                                       sem_act.at[0])
            fp = pltpu.make_async_copy(row0_pos, pos_out.at[pl.ds(off, T)],
                                       sem_pos.at[0])
            fr = pltpu.make_async_copy(row0_rot, rot_out.at[pl.ds(off, T)],
                                       sem_rot.at[0])
            fa.start()
            fp.start()
            fr.start()
            fa.wait()
            fp.wait()
            fr.wait()
            return carry

        lax.fori_loop(0, (M - start) // T, fill, 0)


def kernel(states, W1, b1, W2, b2):
    B, N, FT = states.shape
    F127 = FT - 3
    M = B * N
    R = N
    G = B
    H = W1.shape[1]
    OUT = W2.shape[1]

    flat = states.reshape(M, FT)
    W1z = W1[:H].at[H - 1].set(0.0)
    Wtail = W1[H - 1:H + 1]
    b1r = b1.reshape(1, H)
    b2r = b2.reshape(1, OUT)
    m6 = jnp.array([[1., 0., 0., 0., 0., 0.],
                    [-1., 0., 0., 0., 0., 0.],
                    [0., 1., 0., 0., 0., 0.],
                    [0., 1., 0., 0., 0., 0.],
                    [0., 0., 1., 0., 0., 1.],
                    [0., 0., 0., -1., 1., 0.]], dtype=jnp.float32)

    grid_spec = pltpu.PrefetchScalarGridSpec(
        num_scalar_prefetch=0,
        grid=(G,),
        in_specs=[
            pl.BlockSpec((R, FT), lambda i: (i, 0)),
            pl.BlockSpec((H, H), lambda i: (0, 0)),
            pl.BlockSpec((2, H), lambda i: (0, 0)),
            pl.BlockSpec((1, H), lambda i: (0, 0)),
            pl.BlockSpec((H, OUT), lambda i: (0, 0)),
            pl.BlockSpec((1, OUT), lambda i: (0, 0)),
            pl.BlockSpec((6, 6), lambda i: (0, 0)),
        ],
        out_specs=[
            pl.BlockSpec(memory_space=pl.ANY),
            pl.BlockSpec(memory_space=pl.ANY),
            pl.BlockSpec(memory_space=pl.ANY),
            pl.BlockSpec((R, 1), lambda i: (i, 0)),
        ],
        scratch_shapes=[
            pltpu.VMEM((NS, R + T, OUT), jnp.float32),
            pltpu.VMEM((NS, R + T, 2), jnp.float32),
            pltpu.VMEM((NS, R + T, 4), jnp.float32),
            pltpu.VMEM((T, OUT), jnp.float32),
            pltpu.VMEM((T, 2), jnp.float32),
            pltpu.VMEM((T, 4), jnp.float32),
            pltpu.VMEM((T, OUT), jnp.float32),
            pltpu.VMEM((T, 2), jnp.float32),
            pltpu.VMEM((T, 4), jnp.float32),
            pltpu.SMEM((1,), jnp.int32),
            pltpu.SMEM((NS,), jnp.int32),
            pltpu.SMEM((NS,), jnp.int32),
            pltpu.SemaphoreType.DMA((NS,)),
            pltpu.SemaphoreType.DMA((NS,)),
            pltpu.SemaphoreType.DMA((NS,)),
            pltpu.SemaphoreType.DMA,
        ],
    )

    act, pos, rot, maskf = pl.pallas_call(
        functools.partial(_body, R=R, G=G, M=M, F127=F127),
        grid_spec=grid_spec,
        out_shape=[
            jax.ShapeDtypeStruct((M, OUT), jnp.float32),
            jax.ShapeDtypeStruct((M, 2), jnp.float32),
            jax.ShapeDtypeStruct((M, 4), jnp.float32),
            jax.ShapeDtypeStruct((M, 1), jnp.float32),
        ],
    )(flat, W1z, Wtail, b1r, W2, b2r, m6)

    action_preds = act.reshape(M, OUT // 2, 2)
    ori_pos = pos.reshape(M, 1, 2)
    rotate = rot.reshape(M, 2, 2)
    agent_mask = maskf.reshape(B, N) != 0.0
    return (action_preds, ori_pos, rotate, agent_mask)


# speedup vs baseline: 1.1829x; 1.1829x over previous
import functools

import jax
import jax.numpy as jnp
from jax import lax
from jax.experimental import pallas as pl
from jax.experimental.pallas import tpu as pltpu

NS = 4
T = 8


def _body(x_ref, w1_ref, wt_ref, b1_ref, w2_ref, b2_ref,
          act_out, pos_out, rot_out, mask_out,
          act_b, pos_b, rot_b,
          carry_act, carry_pos, carry_rot,
          row0_act, row0_pos, row0_rot,
          cnt_ref, offs_ref, pend_ref,
          sem_act, sem_pos, sem_rot, sem_aux,
          *, R, G, M, F127):
    RB = R + T
    i = pl.program_id(0)
    slot = lax.rem(i, NS)

    @pl.when(i == 0)
    def _init():
        cnt_ref[0] = 0
        for s in range(NS):
            pend_ref[s] = 0

    def drain(s, off):
        off = pl.multiple_of(off, T)
        pltpu.make_async_copy(act_b.at[s, pl.ds(0, R)],
                              act_out.at[pl.ds(off, R)], sem_act.at[s]).wait()
        pltpu.make_async_copy(pos_b.at[s, pl.ds(0, R)],
                              pos_out.at[pl.ds(off, R)], sem_pos.at[s]).wait()
        pltpu.make_async_copy(rot_b.at[s, pl.ds(0, R)],
                              rot_out.at[pl.ds(off, R)], sem_rot.at[s]).wait()

    @pl.when(pend_ref[slot] == 1)
    def _reclaim():
        drain(slot, offs_ref[slot])
        pend_ref[slot] = 0

    x = x_ref[...]
    maskf = (x[:, F127:F127 + 1] != 0.0).astype(jnp.float32)
    mask_out[...] = maskf

    goal = x[:, F127 - 2:F127]
    posv = x[:, F127 + 1:F127 + 3]
    diff = goal - posv
    dx = diff[:, 0:1]
    dy = diff[:, 1:2]
    r2 = dx * dx + dy * dy
    inv = lax.rsqrt(r2)
    zero = r2 == 0.0
    inv = jnp.where(zero, 0.0, inv)
    c = jnp.where(zero, 1.0, dx * inv)
    s = jnp.where(zero, 0.0, dy * inv)
    lg0 = c * dx - s * dy
    lg1 = s * dx + c * dy
    lgpair = jnp.concatenate([lg0, lg1], axis=1)
    rotv = jnp.concatenate([c, -s, s, c], axis=1)

    h = jnp.maximum(
        jnp.dot(x[:, :F127 + 1], w1_ref[...],
                preferred_element_type=jnp.float32)
        + jnp.dot(lgpair, wt_ref[...], preferred_element_type=jnp.float32)
        + b1_ref[...], 0.0)
    act = (jnp.dot(h, w2_ref[...], preferred_element_type=jnp.float32)
           + b2_ref[...])

    nblk = jnp.sum(maskf).astype(jnp.int32)
    cnt = cnt_ref[0]
    base = pl.multiple_of((cnt // T) * T, T)
    shift = cnt - base
    cnt2 = cnt + nblk
    base2 = pl.multiple_of((cnt2 // T) * T, T)

    fast = jnp.logical_and(nblk == R, shift == 0)

    @pl.when(fast)
    def _fast():
        act_b[slot, pl.ds(0, R)] = act
        pos_b[slot, pl.ds(0, R)] = posv
        rot_b[slot, pl.ds(0, R)] = rotv

    @pl.when(jnp.logical_not(fast))
    def _pack():
        ii = lax.broadcasted_iota(jnp.int32, (R, R), 0)
        jj = lax.broadcasted_iota(jnp.int32, (R, R), 1)
        eye = (ii == jj).astype(jnp.float32)
        su = (ii < jj).astype(jnp.float32)
        diag_m = eye * jnp.broadcast_to(maskf, (R, R))
        ones8 = jnp.full((8, R), 1.0, dtype=jnp.float32)
        mask8 = jnp.dot(ones8, diag_m,
                        preferred_element_type=jnp.float32)
        cum8 = jnp.dot(mask8, su,
                       preferred_element_type=jnp.float32)
        mask_lane = jnp.broadcast_to(mask8[0:1], (RB, R))
        cum_excl = jnp.broadcast_to(cum8[0:1], (RB, R))
        pio = lax.broadcasted_iota(jnp.int32, (RB, R), 0).astype(jnp.float32)
        shf = shift.astype(jnp.float32)
        P = jnp.where(cum_excl + shf == pio, mask_lane, 0.0)

        cmask = lax.broadcasted_iota(jnp.int32, (T, 1), 0) < shift
        zpad_a = jnp.zeros((R, act.shape[1]), jnp.float32)
        zpad_p = jnp.zeros((R, 2), jnp.float32)
        zpad_r = jnp.zeros((R, 4), jnp.float32)
        ca = jnp.concatenate(
            [jnp.where(cmask, carry_act[...], 0.0), zpad_a], axis=0)
        cp = jnp.concatenate(
            [jnp.where(cmask, carry_pos[...], 0.0), zpad_p], axis=0)
        cr = jnp.concatenate(
            [jnp.where(cmask, carry_rot[...], 0.0), zpad_r], axis=0)
        act_b[slot] = jnp.dot(P, act,
                              preferred_element_type=jnp.float32) + ca
        pos_b[slot] = jnp.dot(P, posv,
                              preferred_element_type=jnp.float32) + cp
        rot_b[slot] = jnp.dot(P, rotv,
                              preferred_element_type=jnp.float32) + cr

    pltpu.make_async_copy(act_b.at[slot, pl.ds(0, R)],
                          act_out.at[pl.ds(base, R)], sem_act.at[slot]).start()
    pltpu.make_async_copy(pos_b.at[slot, pl.ds(0, R)],
                          pos_out.at[pl.ds(base, R)], sem_pos.at[slot]).start()
    pltpu.make_async_copy(rot_b.at[slot, pl.ds(0, R)],
                          rot_out.at[pl.ds(base, R)], sem_rot.at[slot]).start()
    offs_ref[slot] = base
    pend_ref[slot] = 1
    cnt_ref[0] = cnt2

    @pl.when(jnp.logical_not(fast))
    def _after_pack():
        d = pl.multiple_of(base2 - base, T)

        @pl.when(cnt2 - base2 > 0)
        def _extract():
            ea = pltpu.make_async_copy(act_b.at[slot, pl.ds(d, T)], carry_act,
                                       sem_aux)
            ep = pltpu.make_async_copy(pos_b.at[slot, pl.ds(d, T)], carry_pos,
                                       sem_aux)
            er = pltpu.make_async_copy(rot_b.at[slot, pl.ds(d, T)], carry_rot,
                                       sem_aux)
            ea.start()
            ea.wait()
            ep.start()
            ep.wait()
            er.start()
            er.wait()

        drain(slot, base)
        pend_ref[slot] = 0

    @pl.when(i == 0)
    def _save_row0():
        row0_act[...] = jnp.broadcast_to(act[0:1, :], (T, act.shape[1]))
        row0_pos[...] = jnp.broadcast_to(posv[0:1, :], (T, 2))
        row0_rot[...] = jnp.broadcast_to(rotv[0:1, :], (T, 4))

    @pl.when(i == G - 1)
    def _tail():
        for sconst in range(NS):
            @pl.when(pend_ref[sconst] == 1)
            def _d(sconst=sconst):
                drain(sconst, offs_ref[sconst])
                pend_ref[sconst] = 0

        cfinal = cnt_ref[0]
        basef = pl.multiple_of((cfinal // T) * T, T)
        shiftf = cfinal - basef

        @pl.when(shiftf > 0)
        def _flush():
            fmask = lax.broadcasted_iota(jnp.int32, (T, 1), 0) < shiftf
            carry_act[...] = jnp.where(fmask, carry_act[...], row0_act[...])
            carry_pos[...] = jnp.where(fmask, carry_pos[...], row0_pos[...])
            carry_rot[...] = jnp.where(fmask, carry_rot[...], row0_rot[...])
            pltpu.make_async_copy(carry_act, act_out.at[pl.ds(basef, T)],
                                  sem_act.at[0]).start()
            pltpu.make_async_copy(carry_pos, pos_out.at[pl.ds(basef, T)],
                                  sem_pos.at[0]).start()
            pltpu.make_async_copy(carry_rot, rot_out.at[pl.ds(basef, T)],
                                  sem_rot.at[0]).start()
            pltpu.make_async_copy(carry_act, act_out.at[pl.ds(basef, T)],
                                  sem_act.at[0]).wait()
            pltpu.make_async_copy(carry_pos, pos_out.at[pl.ds(basef, T)],
                                  sem_pos.at[0]).wait()
            pltpu.make_async_copy(carry_rot, rot_out.at[pl.ds(basef, T)],
                                  sem_rot.at[0]).wait()

        start = pl.multiple_of(basef + jnp.where(shiftf > 0, T, 0), T)

        def fill(k, carry):
            off = pl.multiple_of(start + k * T, T)
            fa = pltpu.make_async_copy(row0_act, act_out.at[pl.ds(off, T)],
                                       sem_act.at[0])
            fp = pltpu.make_async_copy(row0_pos, pos_out.at[pl.ds(off, T)],
                                       sem_pos.at[0])
            fr = pltpu.make_async_copy(row0_rot, rot_out.at[pl.ds(off, T)],
                                       sem_rot.at[0])
            fa.start()
            fp.start()
            fr.start()
            fa.wait()
            fp.wait()
            fr.wait()
            return carry

        lax.fori_loop(0, (M - start) // T, fill, 0)


def kernel(states, W1, b1, W2, b2):
    B, N, FT = states.shape
    F127 = FT - 3
    M = B * N
    R = 2 * N if M % (2 * N) == 0 and B % 2 == 0 else N
    G = M // R
    H = W1.shape[1]
    OUT = W2.shape[1]

    flat = states.reshape(M, FT)
    W1z = W1[:H].at[H - 1].set(0.0)
    Wtail = W1[H - 1:H + 1]
    b1r = b1.reshape(1, H)
    b2r = b2.reshape(1, OUT)

    grid_spec = pltpu.PrefetchScalarGridSpec(
        num_scalar_prefetch=0,
        grid=(G,),
        in_specs=[
            pl.BlockSpec((R, FT), lambda i: (i, 0)),
            pl.BlockSpec((H, H), lambda i: (0, 0)),
            pl.BlockSpec((2, H), lambda i: (0, 0)),
            pl.BlockSpec((1, H), lambda i: (0, 0)),
            pl.BlockSpec((H, OUT), lambda i: (0, 0)),
            pl.BlockSpec((1, OUT), lambda i: (0, 0)),
        ],
        out_specs=[
            pl.BlockSpec(memory_space=pl.ANY),
            pl.BlockSpec(memory_space=pl.ANY),
            pl.BlockSpec(memory_space=pl.ANY),
            pl.BlockSpec((R, 1), lambda i: (i, 0)),
        ],
        scratch_shapes=[
            pltpu.VMEM((NS, R + T, OUT), jnp.float32),
            pltpu.VMEM((NS, R + T, 2), jnp.float32),
            pltpu.VMEM((NS, R + T, 4), jnp.float32),
            pltpu.VMEM((T, OUT), jnp.float32),
            pltpu.VMEM((T, 2), jnp.float32),
            pltpu.VMEM((T, 4), jnp.float32),
            pltpu.VMEM((T, OUT), jnp.float32),
            pltpu.VMEM((T, 2), jnp.float32),
            pltpu.VMEM((T, 4), jnp.float32),
            pltpu.SMEM((1,), jnp.int32),
            pltpu.SMEM((NS,), jnp.int32),
            pltpu.SMEM((NS,), jnp.int32),
            pltpu.SemaphoreType.DMA((NS,)),
            pltpu.SemaphoreType.DMA((NS,)),
            pltpu.SemaphoreType.DMA((NS,)),
            pltpu.SemaphoreType.DMA,
        ],
    )

    act, pos, rot, maskf = pl.pallas_call(
        functools.partial(_body, R=R, G=G, M=M, F127=F127),
        grid_spec=grid_spec,
        out_shape=[
            jax.ShapeDtypeStruct((M, OUT), jnp.float32),
            jax.ShapeDtypeStruct((M, 2), jnp.float32),
            jax.ShapeDtypeStruct((M, 4), jnp.float32),
            jax.ShapeDtypeStruct((M, 1), jnp.float32),
        ],
    )(flat, W1z, Wtail, b1r, W2, b2r)

    action_preds = act.reshape(M, OUT // 2, 2)
    ori_pos = pos.reshape(M, 1, 2)
    rotate = rot.reshape(M, 2, 2)
    agent_mask = maskf.reshape(B, N) != 0.0
    return (action_preds, ori_pos, rotate, agent_mask)
